# baseline (speedup 1.0000x reference)
import functools

import jax
import jax.numpy as jnp
from jax import lax
from jax.experimental import pallas as pl
from jax.experimental.pallas import tpu as pltpu
from jax.experimental.pallas import tpu_sc as plsc

_TOPK = 6
_DIST_THRESH = 120.0
_L = 16
_NEG_INF = float("-inf")


@functools.partial(jax.jit, static_argnums=(5, 6, 7))
def _sc_topk(state_r, lane_r, dist_r, occ_pad, ego_cat, B, N, T):
    npad = ((N + _L - 1) // _L) * _L
    nchunks = npad // _L
    try:
        info = plsc.get_sparse_core_info()
        NC, NS = info.num_cores, info.num_subcores
    except ValueError:
        NC, NS = 2, 16
    NW = NC * NS
    rows_per_w = B // NW
    mesh = plsc.VectorSubcoreMesh(core_axis_name="c", subcore_axis_name="s",
                                  num_cores=NC, num_subcores=NS)

    @functools.partial(
        pl.kernel,
        out_type=[
            jax.ShapeDtypeStruct((B, _L), jnp.float32),
            jax.ShapeDtypeStruct((B, _L), jnp.int32),
            jax.ShapeDtypeStruct((B, _L), jnp.int32),
        ],
        mesh=mesh,
        compiler_params=pltpu.CompilerParams(use_tc_tiling_on_sc=False,
                                             needs_layout_passes=False),
        scratch_types=[
            pltpu.VMEM((npad, T), jnp.float32),
            pltpu.VMEM((npad, _L), jnp.float32),
            pltpu.VMEM((npad, _L), jnp.float32),
            pltpu.VMEM((npad,), jnp.int32),
            pltpu.VMEM((npad,), jnp.int32),
            pltpu.VMEM((npad,), jnp.int32),
            pltpu.VMEM((_L,), jnp.float32),
            pltpu.VMEM((npad,), jnp.float32),
            pltpu.VMEM((npad,), jnp.float32),
            pltpu.VMEM((_L,), jnp.float32),
            pltpu.VMEM((_L,), jnp.int32),
            pltpu.VMEM((_L,), jnp.int32),
            pltpu.SemaphoreType.DMA,
        ],
    )
    def k(state_hbm, lane_hbm, dist_hbm, occ_hbm, ego_hbm,
          oscore_hbm, oidx_hbm, ovalid_hbm,
          dist_v, state_v, lane_v, idx_v, idx2_v, occ_v, ego_v, score_v, dmin_v,
          os_v, oi_v, ov_v, sem):
        wid = lax.axis_index("s") * NC + lax.axis_index("c")
        base = wid * rows_per_w
        iota = lax.iota(jnp.int32, _L)

        def row_body(i, carry):
            b = base + i
            for c in range(nchunks):
                n0 = c * _L
                lid = b * N + iota + n0
                idx_v[pl.ds(n0, _L)] = lid * 4 + 3
                idx2_v[pl.ds(n0, _L)] = lid
            cps = [
                pltpu.async_copy(dist_hbm.at[b], dist_v.at[pl.ds(0, N)], sem),
                pltpu.async_copy(state_hbm.at[idx_v.at[pl.ds(0, 104)]],
                                 state_v.at[pl.ds(0, 104)], sem),
                pltpu.async_copy(state_hbm.at[idx_v.at[pl.ds(104, 96)]],
                                 state_v.at[pl.ds(104, 96)], sem),
                pltpu.async_copy(lane_hbm.at[idx2_v.at[pl.ds(0, 104)]],
                                 lane_v.at[pl.ds(0, 104)], sem),
                pltpu.async_copy(lane_hbm.at[idx2_v.at[pl.ds(104, 96)]],
                                 lane_v.at[pl.ds(104, 96)], sem),
                pltpu.async_copy(occ_hbm.at[b], occ_v, sem),
                pltpu.async_copy(ego_hbm.at[b], ego_v, sem),
            ]
            for cp in cps:
                cp.wait()

            ex = plsc.load_gather(ego_v, [jnp.full((_L,), 1, jnp.int32)])
            ey = plsc.load_gather(ego_v, [jnp.full((_L,), 2, jnp.int32)])
            ev = plsc.load_gather(ego_v, [jnp.full((_L,), 3, jnp.int32)])
            el = plsc.load_gather(ego_v, [jnp.full((_L,), 4, jnp.int32)])

            anyclose = jnp.zeros((_L,), jnp.bool_)
            for c in range(nchunks):
                n0 = c * _L
                nidx = iota + n0
                occ_b = occ_v[pl.ds(n0, _L)] != 0
                dmin = plsc.load_gather(dist_v, [nidx, jnp.zeros((_L,), jnp.int32)])
                for t in range(1, T):
                    col = jnp.full((_L,), t, jnp.int32)
                    dmin = jnp.minimum(dmin, plsc.load_gather(dist_v, [nidx, col]))
                x = plsc.load_gather(state_v, [nidx, jnp.full((_L,), 12, jnp.int32)])
                y = plsc.load_gather(state_v, [nidx, jnp.full((_L,), 13, jnp.int32)])
                v = plsc.load_gather(state_v, [nidx, jnp.full((_L,), 14, jnp.int32)])
                ln = plsc.load_gather(lane_v, [nidx, jnp.full((_L,), T - 1, jnp.int32)])
                ald = jnp.abs(ln - el)
                same = jnp.where(ald < 0.5, 1.0, 0.0).astype(jnp.float32)
                adj = jnp.where(jnp.abs(ald - 1.0) < 0.5, 1.0, 0.0).astype(jnp.float32)
                dx = jnp.abs(x - ex)
                dy = jnp.abs(y - ey)
                closing = jnp.maximum(ev - v, 0.0)
                sc = (1.2 / (dy + 1.0) + 0.9 / (dmin + 1.0)
                      + 0.35 * jnp.minimum(closing * 0.1, 2.0)
                      + 0.25 * same + 0.1 * adj + 0.15 / (dx + 1.0))
                anyclose = anyclose | (occ_b & (dmin <= _DIST_THRESH))
                score_v[pl.ds(n0, _L)] = sc
                dmin_v[pl.ds(n0, _L)] = dmin

            hc = jnp.full((_L,), jnp.any(anyclose))
            ninf = jnp.full((_L,), _NEG_INF, jnp.float32)
            for c in range(nchunks):
                n0 = c * _L
                occ_b = occ_v[pl.ds(n0, _L)] != 0
                close = occ_b & (dmin_v[pl.ds(n0, _L)] <= _DIST_THRESH)
                avail = jnp.where(hc, close, occ_b)
                score_v[pl.ds(n0, _L)] = jnp.where(avail, score_v[pl.ds(n0, _L)], ninf)

            res_s = jnp.zeros((_L,), jnp.float32)
            res_i = jnp.zeros((_L,), jnp.int32)
            res_v = jnp.zeros((_L,), jnp.int32)
            last_inf = jnp.int32(-1)
            for kk in range(_TOPK):
                macc = score_v[pl.ds(0, _L)]
                for c in range(1, nchunks):
                    macc = jnp.maximum(macc, score_v[pl.ds(c * _L, _L)])
                m = jnp.max(macc)
                m_fin = m > _NEG_INF
                fin_b = jnp.full((_L,), m_fin)
                chosen = jnp.int32(0)
                found = jnp.bool_(False)
                for c in range(nchunks):
                    n0 = c * _L
                    nidx = iota + n0
                    allow = (score_v[pl.ds(n0, _L)] == m) & (fin_b | (nidx > last_inf))
                    ffs = plsc.all_reduce_ffs(allow)
                    ffs_s = jnp.max(ffs) if ffs.ndim else ffs
                    has = jnp.any(allow)
                    cand = n0 + ffs_s
                    chosen = jnp.where(found, chosen, jnp.where(has, cand, chosen))
                    found = found | has
                plsc.store_scatter(score_v, [jnp.full((_L,), chosen)],
                                   ninf, mask=iota == 0)
                last_inf = jnp.where(m_fin, last_inf, chosen)
                lane_k = iota == kk
                res_s = jnp.where(lane_k, jnp.full((_L,), m), res_s)
                res_i = jnp.where(lane_k, jnp.full((_L,), chosen), res_i)
                res_v = jnp.where(lane_k, jnp.full((_L,), m_fin.astype(jnp.int32)), res_v)

            os_v[...] = res_s
            oi_v[...] = res_i
            ov_v[...] = res_v
            pltpu.sync_copy(os_v, oscore_hbm.at[b])
            pltpu.sync_copy(oi_v, oidx_hbm.at[b])
            pltpu.sync_copy(ov_v, ovalid_hbm.at[b])
            return carry

        lax.fori_loop(0, rows_per_w, row_body, 0)

    return k(state_r, lane_r, dist_r, occ_pad, ego_cat)


def kernel(ego_state_raw, nbr_state_raw_grid, ego_lane, nbr_lane_grid,
           nbr_dist_grid, social_occ):
    B, N, T, C = nbr_state_raw_grid.shape
    state_r = nbr_state_raw_grid.reshape(B * N * 4, _L)
    lane_r = nbr_lane_grid.reshape(B * N, T)
    dist_r = nbr_dist_grid.reshape(B, N, T)
    npad = ((N + _L - 1) // _L) * _L
    occ_pad = jnp.pad(social_occ.astype(jnp.int32), ((0, 0), (0, npad - N)))
    ego_cat = jnp.concatenate(
        [jnp.zeros((B, 1), jnp.float32), ego_state_raw[:, -1, :3],
         ego_lane[:, -1, :]], axis=-1)
    ego_cat = jnp.pad(ego_cat, ((0, 0), (0, _L - ego_cat.shape[-1])))
    s16, i16, v16 = _sc_topk(state_r, lane_r, dist_r, occ_pad, ego_cat, B, N, T)
    topk_score = s16[:, :_TOPK]
    topk_idx = i16[:, :_TOPK].astype(jnp.int64)
    topk_valid = v16[:, :_TOPK] != 0
    return topk_score, topk_idx, topk_valid


# speedup vs baseline: 7.0677x; 7.0677x over previous
import functools

import jax
import jax.numpy as jnp
from jax import lax
from jax.experimental import pallas as pl
from jax.experimental.pallas import tpu as pltpu
from jax.experimental.pallas import tpu_sc as plsc

_TOPK = 6
_DIST_THRESH = 120.0
_L = 16
_NEG_INF = float("-inf")


@functools.partial(jax.jit, static_argnums=(5, 6, 7))
def _sc_topk(state_r, lane_r, dist_r, occ_pad, ego_cat, B, N, T):
    npad = ((N + _L - 1) // _L) * _L
    nchunks = npad // _L
    try:
        info = plsc.get_sparse_core_info()
        NC, NS = info.num_cores, info.num_subcores
    except ValueError:
        NC, NS = 2, 16
    NW = NC * NS
    rows_per_w = B // NW
    mesh = plsc.VectorSubcoreMesh(core_axis_name="c", subcore_axis_name="s",
                                  num_cores=NC, num_subcores=NS)

    @functools.partial(
        pl.kernel,
        out_type=[
            jax.ShapeDtypeStruct((B, _L), jnp.float32),
            jax.ShapeDtypeStruct((B, _L), jnp.int32),
            jax.ShapeDtypeStruct((B, _L), jnp.int32),
        ],
        mesh=mesh,
        compiler_params=pltpu.CompilerParams(use_tc_tiling_on_sc=False,
                                             needs_layout_passes=False),
        scratch_types=[
            pltpu.VMEM((npad, T), jnp.float32),
            pltpu.VMEM((npad, _L), jnp.float32),
            pltpu.VMEM((npad, _L), jnp.float32),
            pltpu.VMEM((npad,), jnp.int32),
            pltpu.VMEM((_L,), jnp.float32),
            pltpu.VMEM((npad,), jnp.float32),
            pltpu.VMEM((npad,), jnp.float32),
            pltpu.VMEM((_L,), jnp.float32),
            pltpu.VMEM((_L,), jnp.int32),
            pltpu.VMEM((_L,), jnp.int32),
            pltpu.SemaphoreType.DMA,
        ],
    )
    def k(state_hbm, lane_hbm, dist_hbm, occ_hbm, ego_hbm,
          oscore_hbm, oidx_hbm, ovalid_hbm,
          dist_v, state_v, lane_v, occ_v, ego_v, score_v, dmin_v,
          os_v, oi_v, ov_v, sem):
        wid = lax.axis_index("s") * NC + lax.axis_index("c")
        base = wid * rows_per_w
        iota = lax.iota(jnp.int32, _L)

        def row_body(i, carry):
            b = base + i
            cps = [
                pltpu.async_copy(dist_hbm.at[b], dist_v.at[pl.ds(0, N)], sem),
                pltpu.async_copy(state_hbm.at[b, :, 3],
                                 state_v.at[pl.ds(0, N)], sem),
                pltpu.async_copy(lane_hbm.at[b], lane_v.at[pl.ds(0, N)], sem),
                pltpu.async_copy(occ_hbm.at[b], occ_v, sem),
                pltpu.async_copy(ego_hbm.at[b], ego_v, sem),
            ]
            for cp in cps:
                cp.wait()

            ex = plsc.load_gather(ego_v, [jnp.full((_L,), 1, jnp.int32)])
            ey = plsc.load_gather(ego_v, [jnp.full((_L,), 2, jnp.int32)])
            ev = plsc.load_gather(ego_v, [jnp.full((_L,), 3, jnp.int32)])
            el = plsc.load_gather(ego_v, [jnp.full((_L,), 4, jnp.int32)])

            anyclose = jnp.zeros((_L,), jnp.bool_)
            for c in range(nchunks):
                n0 = c * _L
                nidx = iota + n0
                occ_b = occ_v[pl.ds(n0, _L)] != 0
                dmin = plsc.load_gather(dist_v, [nidx, jnp.zeros((_L,), jnp.int32)])
                for t in range(1, T):
                    col = jnp.full((_L,), t, jnp.int32)
                    dmin = jnp.minimum(dmin, plsc.load_gather(dist_v, [nidx, col]))
                x = plsc.load_gather(state_v, [nidx, jnp.full((_L,), 12, jnp.int32)])
                y = plsc.load_gather(state_v, [nidx, jnp.full((_L,), 13, jnp.int32)])
                v = plsc.load_gather(state_v, [nidx, jnp.full((_L,), 14, jnp.int32)])
                ln = plsc.load_gather(lane_v, [nidx, jnp.full((_L,), T - 1, jnp.int32)])
                ald = jnp.abs(ln - el)
                same = jnp.where(ald < 0.5, 1.0, 0.0).astype(jnp.float32)
                adj = jnp.where(jnp.abs(ald - 1.0) < 0.5, 1.0, 0.0).astype(jnp.float32)
                dx = jnp.abs(x - ex)
                dy = jnp.abs(y - ey)
                closing = jnp.maximum(ev - v, 0.0)
                sc = (1.2 / (dy + 1.0) + 0.9 / (dmin + 1.0)
                      + 0.35 * jnp.minimum(closing * 0.1, 2.0)
                      + 0.25 * same + 0.1 * adj + 0.15 / (dx + 1.0))
                anyclose = anyclose | (occ_b & (dmin <= _DIST_THRESH))
                score_v[pl.ds(n0, _L)] = sc
                dmin_v[pl.ds(n0, _L)] = dmin

            hc = jnp.full((_L,), jnp.any(anyclose))
            ninf = jnp.full((_L,), _NEG_INF, jnp.float32)
            for c in range(nchunks):
                n0 = c * _L
                occ_b = occ_v[pl.ds(n0, _L)] != 0
                close = occ_b & (dmin_v[pl.ds(n0, _L)] <= _DIST_THRESH)
                avail = jnp.where(hc, close, occ_b)
                score_v[pl.ds(n0, _L)] = jnp.where(avail, score_v[pl.ds(n0, _L)], ninf)

            res_s = jnp.zeros((_L,), jnp.float32)
            res_i = jnp.zeros((_L,), jnp.int32)
            res_v = jnp.zeros((_L,), jnp.int32)
            last_inf = jnp.int32(-1)
            for kk in range(_TOPK):
                macc = score_v[pl.ds(0, _L)]
                for c in range(1, nchunks):
                    macc = jnp.maximum(macc, score_v[pl.ds(c * _L, _L)])
                m = jnp.max(macc)
                m_fin = m > _NEG_INF
                fin_b = jnp.full((_L,), m_fin)
                chosen = jnp.int32(0)
                found = jnp.bool_(False)
                for c in range(nchunks):
                    n0 = c * _L
                    nidx = iota + n0
                    allow = (score_v[pl.ds(n0, _L)] == m) & (fin_b | (nidx > last_inf))
                    ffs = plsc.all_reduce_ffs(allow)
                    ffs_s = jnp.max(ffs) if ffs.ndim else ffs
                    has = jnp.any(allow)
                    cand = n0 + ffs_s
                    chosen = jnp.where(found, chosen, jnp.where(has, cand, chosen))
                    found = found | has
                plsc.store_scatter(score_v, [jnp.full((_L,), chosen)],
                                   ninf, mask=iota == 0)
                last_inf = jnp.where(m_fin, last_inf, chosen)
                lane_k = iota == kk
                res_s = jnp.where(lane_k, jnp.full((_L,), m), res_s)
                res_i = jnp.where(lane_k, jnp.full((_L,), chosen), res_i)
                res_v = jnp.where(lane_k, jnp.full((_L,), m_fin.astype(jnp.int32)), res_v)

            os_v[...] = res_s
            oi_v[...] = res_i
            ov_v[...] = res_v
            pltpu.sync_copy(os_v, oscore_hbm.at[b])
            pltpu.sync_copy(oi_v, oidx_hbm.at[b])
            pltpu.sync_copy(ov_v, ovalid_hbm.at[b])
            return carry

        lax.fori_loop(0, rows_per_w, row_body, 0)

    return k(state_r, lane_r, dist_r, occ_pad, ego_cat)


def kernel(ego_state_raw, nbr_state_raw_grid, ego_lane, nbr_lane_grid,
           nbr_dist_grid, social_occ):
    B, N, T, C = nbr_state_raw_grid.shape
    state_r = nbr_state_raw_grid.reshape(B, N, 4, _L)
    lane_r = nbr_lane_grid.reshape(B, N, T)
    dist_r = nbr_dist_grid.reshape(B, N, T)
    npad = ((N + _L - 1) // _L) * _L
    occ_pad = jnp.pad(social_occ.astype(jnp.int32), ((0, 0), (0, npad - N)))
    ego_cat = jnp.concatenate(
        [jnp.zeros((B, 1), jnp.float32), ego_state_raw[:, -1, :3],
         ego_lane[:, -1, :]], axis=-1)
    ego_cat = jnp.pad(ego_cat, ((0, 0), (0, _L - ego_cat.shape[-1])))
    s16, i16, v16 = _sc_topk(state_r, lane_r, dist_r, occ_pad, ego_cat, B, N, T)
    topk_score = s16[:, :_TOPK]
    topk_idx = i16[:, :_TOPK].astype(jnp.int64)
    topk_valid = v16[:, :_TOPK] != 0
    return topk_score, topk_idx, topk_valid


# speedup vs baseline: 11.0021x; 1.5567x over previous
import functools

import jax
import jax.numpy as jnp
from jax import lax
from jax.experimental import pallas as pl
from jax.experimental.pallas import tpu as pltpu
from jax.experimental.pallas import tpu_sc as plsc

_TOPK = 6
_DIST_THRESH = 120.0
_L = 16
_NEG_INF = float("-inf")


@functools.partial(jax.jit, static_argnums=(5, 6, 7))
def _sc_topk(state_r, lane_r, dist_r, occ_pad, ego_cat, B, N, T):
    npad = ((N + _L - 1) // _L) * _L
    nchunks = npad // _L
    try:
        info = plsc.get_sparse_core_info()
        NC, NS = info.num_cores, info.num_subcores
    except ValueError:
        NC, NS = 2, 16
    NW = NC * NS
    rows_per_w = B // NW
    mesh = plsc.VectorSubcoreMesh(core_axis_name="c", subcore_axis_name="s",
                                  num_cores=NC, num_subcores=NS)

    @functools.partial(
        pl.kernel,
        out_type=[jax.ShapeDtypeStruct((B, 3 * _L), jnp.int32)],
        mesh=mesh,
        compiler_params=pltpu.CompilerParams(use_tc_tiling_on_sc=False,
                                             needs_layout_passes=False),
        scratch_types=[
            pltpu.VMEM((npad, T), jnp.float32),
            pltpu.VMEM((npad, T), jnp.float32),
            pltpu.VMEM((npad, 4 * T), jnp.float32),
            pltpu.VMEM((npad, 4 * T), jnp.float32),
            pltpu.VMEM((npad, T), jnp.float32),
            pltpu.VMEM((npad, T), jnp.float32),
            pltpu.VMEM((rows_per_w, npad), jnp.int32),
            pltpu.VMEM((rows_per_w, _L), jnp.float32),
            pltpu.VMEM((npad,), jnp.float32),
            pltpu.VMEM((npad,), jnp.float32),
            pltpu.VMEM((3 * _L,), jnp.int32),
            pltpu.SemaphoreType.DMA,
            pltpu.SemaphoreType.DMA,
            pltpu.SemaphoreType.DMA,
        ],
    )
    def k(state_hbm, lane_hbm, dist_hbm, occ_hbm, ego_hbm, out_hbm,
          dist_a, dist_b, state_a, state_b, lane_a, lane_b,
          occ_v, ego_v, score_v, dmin_v, pack_v, sem_a, sem_b, sem_s):
        wid = lax.axis_index("s") * NC + lax.axis_index("c")
        base = wid * rows_per_w
        iota = lax.iota(jnp.int32, _L)

        def issue(b, dv, sv, lv, sem):
            pltpu.async_copy(dist_hbm.at[b], dv.at[pl.ds(0, N)], sem)
            pltpu.async_copy(state_hbm.at[b], sv.at[pl.ds(0, N)], sem)
            pltpu.async_copy(lane_hbm.at[b], lv.at[pl.ds(0, N)], sem)

        def drain(b, dv, sv, lv, sem):
            pltpu.make_async_copy(dist_hbm.at[b], dv.at[pl.ds(0, N)], sem).wait()
            pltpu.make_async_copy(state_hbm.at[b], sv.at[pl.ds(0, N)], sem).wait()
            pltpu.make_async_copy(lane_hbm.at[b], lv.at[pl.ds(0, N)], sem).wait()

        def compute_row(b, r, dist_v, state_v, lane_v):
            last0 = (T - 1) * 4
            rfull = jnp.full((_L,), r, jnp.int32)
            ex = plsc.load_gather(ego_v, [rfull, jnp.full((_L,), 1, jnp.int32)])
            ey = plsc.load_gather(ego_v, [rfull, jnp.full((_L,), 2, jnp.int32)])
            ev = plsc.load_gather(ego_v, [rfull, jnp.full((_L,), 3, jnp.int32)])
            el = plsc.load_gather(ego_v, [rfull, jnp.full((_L,), 4, jnp.int32)])

            anyclose = jnp.zeros((_L,), jnp.bool_)
            for c in range(nchunks):
                n0 = c * _L
                nidx = iota + n0
                occ_b = occ_v[r, pl.ds(n0, _L)] != 0
                dmin = plsc.load_gather(dist_v, [nidx, jnp.zeros((_L,), jnp.int32)])
                for t in range(1, T):
                    col = jnp.full((_L,), t, jnp.int32)
                    dmin = jnp.minimum(dmin, plsc.load_gather(dist_v, [nidx, col]))
                x = plsc.load_gather(state_v, [nidx, jnp.full((_L,), last0, jnp.int32)])
                y = plsc.load_gather(state_v, [nidx, jnp.full((_L,), last0 + 1, jnp.int32)])
                v = plsc.load_gather(state_v, [nidx, jnp.full((_L,), last0 + 2, jnp.int32)])
                ln = plsc.load_gather(lane_v, [nidx, jnp.full((_L,), T - 1, jnp.int32)])
                ald = jnp.abs(ln - el)
                same = jnp.where(ald < 0.5, 1.0, 0.0).astype(jnp.float32)
                adj = jnp.where(jnp.abs(ald - 1.0) < 0.5, 1.0, 0.0).astype(jnp.float32)
                dx = jnp.abs(x - ex)
                dy = jnp.abs(y - ey)
                closing = jnp.maximum(ev - v, 0.0)
                sc = (1.2 / (dy + 1.0) + 0.9 / (dmin + 1.0)
                      + 0.35 * jnp.minimum(closing * 0.1, 2.0)
                      + 0.25 * same + 0.1 * adj + 0.15 / (dx + 1.0))
                anyclose = anyclose | (occ_b & (dmin <= _DIST_THRESH))
                score_v[pl.ds(n0, _L)] = sc
                dmin_v[pl.ds(n0, _L)] = dmin

            hc = jnp.full((_L,), jnp.any(anyclose))
            ninf = jnp.full((_L,), _NEG_INF, jnp.float32)
            for c in range(nchunks):
                n0 = c * _L
                occ_b = occ_v[r, pl.ds(n0, _L)] != 0
                close = occ_b & (dmin_v[pl.ds(n0, _L)] <= _DIST_THRESH)
                avail = jnp.where(hc, close, occ_b)
                score_v[pl.ds(n0, _L)] = jnp.where(avail, score_v[pl.ds(n0, _L)], ninf)

            res_s = jnp.zeros((_L,), jnp.float32)
            res_i = jnp.zeros((_L,), jnp.int32)
            res_v = jnp.zeros((_L,), jnp.int32)
            last_inf = jnp.int32(-1)
            for kk in range(_TOPK):
                macc = score_v[pl.ds(0, _L)]
                for c in range(1, nchunks):
                    macc = jnp.maximum(macc, score_v[pl.ds(c * _L, _L)])
                m = jnp.max(macc)
                m_fin = m > _NEG_INF
                fin_b = jnp.full((_L,), m_fin)
                chosen = jnp.int32(0)
                found = jnp.bool_(False)
                for c in range(nchunks):
                    n0 = c * _L
                    nidx = iota + n0
                    allow = (score_v[pl.ds(n0, _L)] == m) & (fin_b | (nidx > last_inf))
                    ffs = plsc.all_reduce_ffs(allow)
                    ffs_s = jnp.max(ffs) if ffs.ndim else ffs
                    has = jnp.any(allow)
                    cand = n0 + ffs_s
                    chosen = jnp.where(found, chosen, jnp.where(has, cand, chosen))
                    found = found | has
                plsc.store_scatter(score_v, [jnp.full((_L,), chosen)],
                                   ninf, mask=iota == 0)
                last_inf = jnp.where(m_fin, last_inf, chosen)
                lane_k = iota == kk
                res_s = jnp.where(lane_k, jnp.full((_L,), m), res_s)
                res_i = jnp.where(lane_k, jnp.full((_L,), chosen), res_i)
                res_v = jnp.where(lane_k, jnp.full((_L,), m_fin.astype(jnp.int32)), res_v)

            pack_v[pl.ds(0, _L)] = plsc.bitcast(res_s, jnp.int32)
            pack_v[pl.ds(_L, _L)] = res_i
            pack_v[pl.ds(2 * _L, _L)] = res_v
            pltpu.sync_copy(pack_v, out_hbm.at[b])

        pltpu.async_copy(occ_hbm.at[pl.ds(base, rows_per_w)], occ_v, sem_s)
        pltpu.async_copy(ego_hbm.at[pl.ds(base, rows_per_w)], ego_v, sem_s)
        pltpu.make_async_copy(occ_hbm.at[pl.ds(base, rows_per_w)], occ_v, sem_s).wait()
        pltpu.make_async_copy(ego_hbm.at[pl.ds(base, rows_per_w)], ego_v, sem_s).wait()

        issue(base, dist_a, state_a, lane_a, sem_a)

        def pair_body(i, carry):
            ba = base + 2 * i
            bb = ba + 1
            issue(bb, dist_b, state_b, lane_b, sem_b)
            drain(ba, dist_a, state_a, lane_a, sem_a)
            compute_row(ba, 2 * i, dist_a, state_a, lane_a)

            @pl.when(2 * i + 2 < rows_per_w)
            def _():
                issue(ba + 2, dist_a, state_a, lane_a, sem_a)

            drain(bb, dist_b, state_b, lane_b, sem_b)
            compute_row(bb, 2 * i + 1, dist_b, state_b, lane_b)
            return carry

        lax.fori_loop(0, rows_per_w // 2, pair_body, 0)

    return k(state_r, lane_r, dist_r, occ_pad, ego_cat)[0]


def kernel(ego_state_raw, nbr_state_raw_grid, ego_lane, nbr_lane_grid,
           nbr_dist_grid, social_occ):
    B, N, T, C = nbr_state_raw_grid.shape
    state_r = nbr_state_raw_grid.reshape(B, N, T * C)
    lane_r = nbr_lane_grid.reshape(B, N, T)
    dist_r = nbr_dist_grid.reshape(B, N, T)
    npad = ((N + _L - 1) // _L) * _L
    occ_pad = jnp.pad(social_occ.astype(jnp.int32), ((0, 0), (0, npad - N)))
    ego_cat = jnp.concatenate(
        [jnp.zeros((B, 1), jnp.float32), ego_state_raw[:, -1, :3],
         ego_lane[:, -1, :]], axis=-1)
    ego_cat = jnp.pad(ego_cat, ((0, 0), (0, _L - ego_cat.shape[-1])))
    packed = _sc_topk(state_r, lane_r, dist_r, occ_pad, ego_cat, B, N, T)
    topk_score = jax.lax.bitcast_convert_type(packed[:, :_TOPK], jnp.float32)
    topk_idx = packed[:, _L:_L + _TOPK].astype(jnp.int64)
    topk_valid = packed[:, 2 * _L:2 * _L + _TOPK] != 0
    return topk_score, topk_idx, topk_valid


# speedup vs baseline: 11.3455x; 1.0312x over previous
import functools

import jax
import jax.numpy as jnp
from jax import lax
from jax.experimental import pallas as pl
from jax.experimental.pallas import tpu as pltpu
from jax.experimental.pallas import tpu_sc as plsc

_TOPK = 6
_DIST_THRESH = 120.0
_L = 16
_NEG_INF = float("-inf")


@functools.partial(jax.jit, static_argnums=(5, 6, 7))
def _sc_topk(state_r, lane_r, dist_r, occ_pad, ego_cat, B, N, T):
    npad = ((N + _L - 1) // _L) * _L
    nchunks = npad // _L
    try:
        info = plsc.get_sparse_core_info()
        NC, NS = info.num_cores, info.num_subcores
    except ValueError:
        NC, NS = 2, 16
    NW = NC * NS
    rows_per_w = B // NW
    mesh = plsc.VectorSubcoreMesh(core_axis_name="c", subcore_axis_name="s",
                                  num_cores=NC, num_subcores=NS)

    @functools.partial(
        pl.kernel,
        out_type=[jax.ShapeDtypeStruct((B, 3 * _L), jnp.int32)],
        mesh=mesh,
        compiler_params=pltpu.CompilerParams(use_tc_tiling_on_sc=False,
                                             needs_layout_passes=False),
        scratch_types=[
            pltpu.VMEM((npad, T), jnp.float32),
            pltpu.VMEM((npad, T), jnp.float32),
            pltpu.VMEM((npad, 4 * T), jnp.float32),
            pltpu.VMEM((npad, 4 * T), jnp.float32),
            pltpu.VMEM((npad, T), jnp.float32),
            pltpu.VMEM((npad, T), jnp.float32),
            pltpu.VMEM((rows_per_w, npad), jnp.int32),
            pltpu.VMEM((rows_per_w, _L), jnp.float32),
            pltpu.VMEM((npad,), jnp.float32),
            pltpu.VMEM((npad,), jnp.float32),
            pltpu.VMEM((3 * _L,), jnp.int32),
            pltpu.SemaphoreType.DMA,
            pltpu.SemaphoreType.DMA,
            pltpu.SemaphoreType.DMA,
        ],
    )
    def k(state_hbm, lane_hbm, dist_hbm, occ_hbm, ego_hbm, out_hbm,
          dist_a, dist_b, state_a, state_b, lane_a, lane_b,
          occ_v, ego_v, score_v, dmin_v, pack_v, sem_a, sem_b, sem_s):
        wid = lax.axis_index("s") * NC + lax.axis_index("c")
        base = wid * rows_per_w
        iota = lax.iota(jnp.int32, _L)

        def issue(b, dv, sv, lv, sem):
            pltpu.async_copy(dist_hbm.at[b], dv.at[pl.ds(0, N)], sem)
            pltpu.async_copy(state_hbm.at[b], sv.at[pl.ds(0, N)], sem)
            pltpu.async_copy(lane_hbm.at[b], lv.at[pl.ds(0, N)], sem)

        def drain(b, dv, sv, lv, sem):
            pltpu.make_async_copy(dist_hbm.at[b], dv.at[pl.ds(0, N)], sem).wait()
            pltpu.make_async_copy(state_hbm.at[b], sv.at[pl.ds(0, N)], sem).wait()
            pltpu.make_async_copy(lane_hbm.at[b], lv.at[pl.ds(0, N)], sem).wait()

        def compute_row(b, r, dist_v, state_v, lane_v):
            last0 = (T - 1) * 4
            rfull = jnp.full((_L,), r, jnp.int32)
            ex = plsc.load_gather(ego_v, [rfull, jnp.full((_L,), 1, jnp.int32)])
            ey = plsc.load_gather(ego_v, [rfull, jnp.full((_L,), 2, jnp.int32)])
            ev = plsc.load_gather(ego_v, [rfull, jnp.full((_L,), 3, jnp.int32)])
            el = plsc.load_gather(ego_v, [rfull, jnp.full((_L,), 4, jnp.int32)])

            anyclose = jnp.zeros((_L,), jnp.bool_)
            for c in range(nchunks):
                n0 = c * _L
                nidx = iota + n0
                occ_b = occ_v[r, pl.ds(n0, _L)] != 0
                dmin = plsc.load_gather(dist_v, [nidx, jnp.zeros((_L,), jnp.int32)])
                for t in range(1, T):
                    col = jnp.full((_L,), t, jnp.int32)
                    dmin = jnp.minimum(dmin, plsc.load_gather(dist_v, [nidx, col]))
                x = plsc.load_gather(state_v, [nidx, jnp.full((_L,), last0, jnp.int32)])
                y = plsc.load_gather(state_v, [nidx, jnp.full((_L,), last0 + 1, jnp.int32)])
                v = plsc.load_gather(state_v, [nidx, jnp.full((_L,), last0 + 2, jnp.int32)])
                ln = plsc.load_gather(lane_v, [nidx, jnp.full((_L,), T - 1, jnp.int32)])
                ald = jnp.abs(ln - el)
                same = jnp.where(ald < 0.5, 1.0, 0.0).astype(jnp.float32)
                adj = jnp.where(jnp.abs(ald - 1.0) < 0.5, 1.0, 0.0).astype(jnp.float32)
                dx = jnp.abs(x - ex)
                dy = jnp.abs(y - ey)
                closing = jnp.maximum(ev - v, 0.0)
                sc = (1.2 / (dy + 1.0) + 0.9 / (dmin + 1.0)
                      + 0.35 * jnp.minimum(closing * 0.1, 2.0)
                      + 0.25 * same + 0.1 * adj + 0.15 / (dx + 1.0))
                anyclose = anyclose | (occ_b & (dmin <= _DIST_THRESH))
                score_v[pl.ds(n0, _L)] = sc
                dmin_v[pl.ds(n0, _L)] = dmin

            hc = jnp.full((_L,), jnp.any(anyclose))
            ninf = jnp.full((_L,), _NEG_INF, jnp.float32)
            for c in range(nchunks):
                n0 = c * _L
                occ_b = occ_v[r, pl.ds(n0, _L)] != 0
                close = occ_b & (dmin_v[pl.ds(n0, _L)] <= _DIST_THRESH)
                avail = jnp.where(hc, close, occ_b)
                score_v[pl.ds(n0, _L)] = jnp.where(avail, score_v[pl.ds(n0, _L)], ninf)

            res_s = jnp.zeros((_L,), jnp.float32)
            res_i = jnp.zeros((_L,), jnp.int32)
            res_v = jnp.zeros((_L,), jnp.int32)
            last_inf = jnp.int32(-1)
            for kk in range(_TOPK):
                macc = score_v[pl.ds(0, _L)]
                for c in range(1, nchunks):
                    macc = jnp.maximum(macc, score_v[pl.ds(c * _L, _L)])
                m = jnp.max(macc)
                m_fin = m > _NEG_INF
                fin_b = jnp.full((_L,), m_fin)
                big = jnp.full((_L,), 1 << 20, jnp.int32)
                iacc = big
                for c in range(nchunks):
                    n0 = c * _L
                    nidx = iota + n0
                    allow = (score_v[pl.ds(n0, _L)] == m) & (fin_b | (nidx > last_inf))
                    iacc = jnp.minimum(iacc, jnp.where(allow, nidx, big))
                chosen = jnp.min(iacc)
                plsc.store_scatter(score_v, [jnp.full((_L,), chosen)],
                                   ninf, mask=iota == 0)
                last_inf = jnp.where(m_fin, last_inf, chosen)
                lane_k = iota == kk
                res_s = jnp.where(lane_k, jnp.full((_L,), m), res_s)
                res_i = jnp.where(lane_k, jnp.full((_L,), chosen), res_i)
                res_v = jnp.where(lane_k, jnp.full((_L,), m_fin.astype(jnp.int32)), res_v)

            pack_v[pl.ds(0, _L)] = plsc.bitcast(res_s, jnp.int32)
            pack_v[pl.ds(_L, _L)] = res_i
            pack_v[pl.ds(2 * _L, _L)] = res_v
            pltpu.sync_copy(pack_v, out_hbm.at[b])

        pltpu.async_copy(occ_hbm.at[pl.ds(base, rows_per_w)], occ_v, sem_s)
        pltpu.async_copy(ego_hbm.at[pl.ds(base, rows_per_w)], ego_v, sem_s)
        pltpu.make_async_copy(occ_hbm.at[pl.ds(base, rows_per_w)], occ_v, sem_s).wait()
        pltpu.make_async_copy(ego_hbm.at[pl.ds(base, rows_per_w)], ego_v, sem_s).wait()

        issue(base, dist_a, state_a, lane_a, sem_a)

        def pair_body(i, carry):
            ba = base + 2 * i
            bb = ba + 1
            issue(bb, dist_b, state_b, lane_b, sem_b)
            drain(ba, dist_a, state_a, lane_a, sem_a)
            compute_row(ba, 2 * i, dist_a, state_a, lane_a)

            @pl.when(2 * i + 2 < rows_per_w)
            def _():
                issue(ba + 2, dist_a, state_a, lane_a, sem_a)

            drain(bb, dist_b, state_b, lane_b, sem_b)
            compute_row(bb, 2 * i + 1, dist_b, state_b, lane_b)
            return carry

        lax.fori_loop(0, rows_per_w // 2, pair_body, 0)

    return k(state_r, lane_r, dist_r, occ_pad, ego_cat)[0]


def kernel(ego_state_raw, nbr_state_raw_grid, ego_lane, nbr_lane_grid,
           nbr_dist_grid, social_occ):
    B, N, T, C = nbr_state_raw_grid.shape
    state_r = nbr_state_raw_grid.reshape(B, N, T * C)
    lane_r = nbr_lane_grid.reshape(B, N, T)
    dist_r = nbr_dist_grid.reshape(B, N, T)
    npad = ((N + _L - 1) // _L) * _L
    occ_pad = jnp.pad(social_occ.astype(jnp.int32), ((0, 0), (0, npad - N)))
    ego_cat = jnp.concatenate(
        [jnp.zeros((B, 1), jnp.float32), ego_state_raw[:, -1, :3],
         ego_lane[:, -1, :]], axis=-1)
    ego_cat = jnp.pad(ego_cat, ((0, 0), (0, _L - ego_cat.shape[-1])))
    packed = _sc_topk(state_r, lane_r, dist_r, occ_pad, ego_cat, B, N, T)
    topk_score = jax.lax.bitcast_convert_type(packed[:, :_TOPK], jnp.float32)
    topk_idx = packed[:, _L:_L + _TOPK].astype(jnp.int64)
    topk_valid = packed[:, 2 * _L:2 * _L + _TOPK] != 0
    return topk_score, topk_idx, topk_valid
